# baseline (speedup 1.0000x reference)
import jax
import jax.numpy as jnp
from jax.experimental import pallas as pl
from jax.experimental.pallas import tpu as pltpu


def _copy_body(in_ref, out_ref):
    out_ref[...] = in_ref[...]


def kernel(inputs, transition_params):
    del transition_params
    x = inputs.astype(jnp.float32).reshape(8192, 128)
    y = pl.pallas_call(
        _copy_body,
        out_shape=jax.ShapeDtypeStruct((8192, 128), jnp.float32),
    )(x)
    return y.reshape(inputs.shape)


# speedup vs baseline: 1.0209x; 1.0209x over previous
import jax
import jax.numpy as jnp
from jax import lax
from jax.experimental import pallas as pl
from jax.experimental.pallas import tpu as pltpu
from jax.experimental.pallas import tpu_sc as plsc

_B, _S, _C = 16, 2048, 32
_HALF = _S // 2


def _sc_copy_body(in_hbm, out_hbm, buf, sem):
    nc = plsc.get_sparse_core_info().num_cores
    wid = lax.axis_index("s") * nc + lax.axis_index("c")
    b = wid // 2
    h = wid % 2
    src = in_hbm.at[b, pl.ds(h * _HALF, _HALF)]
    dst = out_hbm.at[b, pl.ds(h * _HALF, _HALF)]
    pltpu.async_copy(src, buf, sem).wait()
    pltpu.async_copy(buf, dst, sem).wait()


def kernel(inputs, transition_params):
    del transition_params
    x = inputs.astype(jnp.float32)
    mesh = plsc.VectorSubcoreMesh(core_axis_name="c", subcore_axis_name="s")
    fn = pl.kernel(
        _sc_copy_body,
        out_type=jax.ShapeDtypeStruct((_B, _S, _C), jnp.float32),
        mesh=mesh,
        scratch_types=[
            pltpu.VMEM((_HALF, _C), jnp.float32),
            pltpu.SemaphoreType.DMA,
        ],
    )
    return fn(x)
